# baseline (speedup 1.0000x reference)
import jax
import jax.numpy as jnp
from jax.experimental import pallas as pl

_VOCAB = 1024
_SEQ_BLK = 2048


def _fill_body(tgt_ref, out_ref):
    b = pl.program_id(0)
    j = pl.program_id(1)
    seq_len = tgt_ref.shape[1]
    s = out_ref.shape[1]

    col = jax.lax.broadcasted_iota(jnp.int32, (s, _VOCAB), 1)
    pos = j * s + jax.lax.broadcasted_iota(jnp.int32, (s, _VOCAB), 0)

    p0 = j * s
    tgt = tgt_ref[b, pl.ds(p0, s)]

    base = jnp.where(
        col == 0, 1.0, jnp.where(col == 1, 0.5, jnp.where(col == 2, 0.0, -1000.0))
    )
    vals = jnp.where(col == tgt[:, None], 5.0, base)
    last_row = jnp.where(col == 0, 0.0, -1000.0)
    vals = jnp.where(pos == seq_len - 1, last_row, vals)
    out_ref[0] = vals.astype(jnp.float32)


def kernel(input_ids):
    bsz, seq_len = input_ids.shape
    loss = jnp.asarray(0.25, dtype=jnp.float32)
    n_blk = seq_len // _SEQ_BLK
    targets = jnp.concatenate(
        [input_ids[:, 1:] % _VOCAB, jnp.full((bsz, 1), -1, jnp.int32)], axis=1
    )
    logits = pl.pallas_call(
        _fill_body,
        grid=(bsz, n_blk),
        in_specs=[pl.BlockSpec((bsz, seq_len), lambda b, j: (0, 0))],
        out_specs=pl.BlockSpec((1, _SEQ_BLK, _VOCAB), lambda b, j: (b, j, 0)),
        out_shape=jax.ShapeDtypeStruct((bsz, seq_len, _VOCAB), jnp.float32),
    )(targets)
    return loss, logits
